# baseline (speedup 1.0000x reference)
import functools

import jax
import jax.numpy as jnp
from jax.experimental import pallas as pl
from jax.experimental.pallas import tpu as pltpu

HEADS = 4
NHID = 256
HC = HEADS * NHID
NGRAPH = 64


def _mm_body(inp_ref, w_ref, a8_ref, b_ref, h_ref, a8o_ref, *, act):
    z = inp_ref[...]
    if act:
        z = z + b_ref[...]
        z = jnp.where(z >= 0, z, 0.01 * z)
    h = jnp.dot(z, w_ref[...], preferred_element_type=jnp.float32)
    h_ref[...] = h
    a8o_ref[...] = jnp.dot(h, a8_ref[...], preferred_element_type=jnp.float32)


def _layer_mm(inp, W, A8, bprev, act):
    N, K = inp.shape
    BN = 1000
    return pl.pallas_call(
        functools.partial(_mm_body, act=act),
        grid=(N // BN,),
        in_specs=[
            pl.BlockSpec((BN, K), lambda i: (i, 0)),
            pl.BlockSpec((K, HC), lambda i: (0, 0)),
            pl.BlockSpec((HC, 8), lambda i: (0, 0)),
            pl.BlockSpec((1, HC), lambda i: (0, 0)),
        ],
        out_specs=[
            pl.BlockSpec((BN, HC), lambda i: (i, 0)),
            pl.BlockSpec((BN, 8), lambda i: (i, 0)),
        ],
        out_shape=[
            jax.ShapeDtypeStruct((N, HC), jnp.float32),
            jax.ShapeDtypeStruct((N, 8), jnp.float32),
        ],
    )(inp, W, A8, bprev)


def _pool_body(h_ref, batch_ref, b3_ref, wm1_ref, bm1_ref, wm2_ref, bm2_ref,
               out_ref, psum, pcnt):
    i = pl.program_id(0)

    @pl.when(i == 0)
    def _init():
        psum[...] = jnp.zeros_like(psum)
        pcnt[...] = jnp.zeros_like(pcnt)

    t = h_ref[...] + b3_ref[...]
    t = jnp.where(t >= 0, t, 0.01 * t)
    b = batch_ref[0, 0, :]
    onehot = (jax.lax.broadcasted_iota(jnp.int32, (NGRAPH, t.shape[0]), 0)
              == b[None, :]).astype(jnp.float32)
    psum[...] += jnp.dot(onehot, t, preferred_element_type=jnp.float32)
    pcnt[...] += jnp.sum(onehot, axis=1, keepdims=True)

    @pl.when(i == pl.num_programs(0) - 1)
    def _fin():
        pooled = psum[...] / jnp.maximum(pcnt[..., 0:1], 1.0)
        r = jnp.dot(pooled, wm1_ref[...], preferred_element_type=jnp.float32)
        r = jnp.maximum(r + bm1_ref[...], 0.0)
        out_ref[...] = (jnp.dot(r, wm2_ref[...],
                                preferred_element_type=jnp.float32)
                        + bm2_ref[...])


def _pool_mlp(h_pad, batch_pad, b3, Wm1, bm1, Wm2, bm2):
    NP = h_pad.shape[0]
    BN = 1024
    NB = NP // BN
    batch3 = batch_pad.reshape(NB, 1, BN)
    NOUT = Wm2.shape[1]
    return pl.pallas_call(
        _pool_body,
        grid=(NB,),
        in_specs=[
            pl.BlockSpec((BN, HC), lambda i: (i, 0)),
            pl.BlockSpec((1, 1, BN), lambda i: (i, 0, 0)),
            pl.BlockSpec((1, HC), lambda i: (0, 0)),
            pl.BlockSpec((HC, NHID), lambda i: (0, 0)),
            pl.BlockSpec((1, NHID), lambda i: (0, 0)),
            pl.BlockSpec((NHID, NOUT), lambda i: (0, 0)),
            pl.BlockSpec((1, NOUT), lambda i: (0, 0)),
        ],
        out_specs=pl.BlockSpec((NGRAPH, NOUT), lambda i: (0, 0)),
        out_shape=jax.ShapeDtypeStruct((NGRAPH, NOUT), jnp.float32),
        scratch_shapes=[
            pltpu.VMEM((NGRAPH, HC), jnp.float32),
            pltpu.VMEM((NGRAPH, 128), jnp.float32),
        ],
    )(h_pad, batch3, b3, Wm1, bm1, Wm2, bm2)


def _edge_phase(h, a8, src_s, dst_s, N):
    a_src = a8[:, :4]
    a_dst = a8[:, 4:]
    e = a_src[src_s] + a_dst[dst_s]
    e = jnp.where(e >= 0, e, 0.2 * e)
    e_max = jax.ops.segment_max(e, dst_s, num_segments=N)
    e_exp = jnp.exp(e - e_max[dst_s])
    denom = jax.ops.segment_sum(e_exp, dst_s, num_segments=N)
    alpha = e_exp / (denom[dst_s] + 1e-16)
    msg = h.reshape(N, HEADS, NHID)[src_s] * alpha[:, :, None]
    out = jax.ops.segment_sum(msg, dst_s, num_segments=N)
    return out.reshape(N, HC)


def _a8(att_s, att_d):
    asf = att_s.reshape(HC)
    adf = att_d.reshape(HC)
    ind = jnp.arange(HC, dtype=jnp.int32) // NHID
    oh = (ind[:, None] == jnp.arange(HEADS, dtype=jnp.int32)[None, :]).astype(jnp.float32)
    return jnp.concatenate([asf[:, None] * oh, adf[:, None] * oh], axis=1)


def kernel(x, edge_index, batch, W1, as1, ad1, b1, W2, as2, ad2, b2,
           W3, as3, ad3, b3, Wm1, bm1, Wm2, bm2):
    N = x.shape[0]
    src = edge_index[0].astype(jnp.int32)
    dst = edge_index[1].astype(jnp.int32)
    loop = jnp.arange(N, dtype=jnp.int32)
    src = jnp.concatenate([src, loop])
    dst = jnp.concatenate([dst, loop])
    order = jnp.argsort(dst)
    src_s = src[order]
    dst_s = dst[order]

    zeros_b = jnp.zeros((1, HC), jnp.float32)
    h1, a81 = _layer_mm(x, W1, _a8(as1, ad1), zeros_b, act=False)
    g1 = _edge_phase(h1, a81, src_s, dst_s, N)

    h2, a82 = _layer_mm(g1, W2, _a8(as2, ad2), b1.reshape(1, HC), act=True)
    g2 = _edge_phase(h2, a82, src_s, dst_s, N)

    h3, a83 = _layer_mm(g2, W3, _a8(as3, ad3), b2.reshape(1, HC), act=True)
    g3 = _edge_phase(h3, a83, src_s, dst_s, N)

    NP = 10240
    g3p = jnp.pad(g3, ((0, NP - N), (0, 0)))
    batch_pad = jnp.pad(batch.astype(jnp.int32), (0, NP - N),
                        constant_values=NGRAPH)
    return _pool_mlp(g3p, batch_pad, b3.reshape(1, HC), Wm1,
                     bm1.reshape(1, NHID), Wm2, bm2.reshape(1, Wm2.shape[1]))
